# Initial kernel scaffold; baseline (speedup 1.0000x reference)
import jax
import jax.numpy as jnp
from jax.experimental import pallas as pl


def kernel(anonymized_nodes, walk_masks, kernel):
    raise NotImplementedError("write your pallas kernel here")


# speedup vs baseline: 3.1384x; 3.1384x over previous
import jax
import jax.numpy as jnp
from jax import lax
from jax.experimental import pallas as pl

B, W, L = 16, 256, 20
J = L * L


def _cooc_kernel(nodes_ref, masks_ref, k_ref, out_ref):
    nodes = nodes_ref[0].astype(jnp.float32)
    masks = masks_ref[0].astype(jnp.float32)
    k = k_ref[...]

    prow = lax.broadcasted_iota(jnp.int32, (L, J), 0)
    jcol = lax.broadcasted_iota(jnp.int32, (L, J), 1)
    S = (prow == jcol // L).astype(jnp.float32)

    vidx = lax.broadcasted_iota(jnp.int32, (W, J), 1) % L

    nrep = jnp.dot(nodes, S, preferred_element_type=jnp.float32)
    mrep = jnp.dot(masks, S, preferred_element_type=jnp.float32)
    X = jnp.where(nrep == vidx.astype(jnp.float32), mrep, 0.0)

    kS = jnp.dot(k, S, preferred_element_type=jnp.float32)
    Krep = lax.dot_general(S, kS, (((0,), (0,)), ((), ())),
                           preferred_element_type=jnp.float32)
    vr = lax.broadcasted_iota(jnp.int32, (J, J), 0) % L
    vc = lax.broadcasted_iota(jnp.int32, (J, J), 1) % L
    Kbig = jnp.where(vr == vc, Krep, 0.0)

    Y = jnp.dot(X, Kbig, preferred_element_type=jnp.float32)
    C = lax.dot_general(Y, X, (((1,), (1,)), ((), ())),
                        preferred_element_type=jnp.float32)

    colsum = jnp.sum(X, axis=0, keepdims=True)
    veqr = lax.broadcasted_iota(jnp.int32, (J, J), 0) % L
    veqc = lax.broadcasted_iota(jnp.int32, (J, J), 1) % L
    Veq = (veqr == veqc).astype(jnp.float32)
    counts_rep = jnp.dot(colsum, Veq,
                         preferred_element_type=jnp.float32)
    eyeL = (lax.broadcasted_iota(jnp.int32, (L, L), 0) ==
            lax.broadcasted_iota(jnp.int32, (L, L), 1)).astype(jnp.float32)
    kdiag = jnp.sum(k * eyeL, axis=0, keepdims=True)
    kdiag_rep = jnp.dot(kdiag, S, preferred_element_type=jnp.float32)
    svec = jnp.where(counts_rep == 1.0, kdiag_rep, 0.0)
    contrib = jnp.sum(X * svec, axis=1, keepdims=True)

    lens_col = jnp.sum(masks, axis=1, keepdims=True)
    ones_row = jnp.ones((1, L), dtype=jnp.float32)
    lens_row = lax.dot_general(ones_row, masks, (((1,), (1,)), ((), ())),
                               preferred_element_type=jnp.float32)

    eyeW = (lax.broadcasted_iota(jnp.int32, (W, W), 0) ==
            lax.broadcasted_iota(jnp.int32, (W, W), 1)).astype(jnp.float32)
    C = C - contrib * eyeW
    norm = lens_col * lens_row + 1e-8
    out_ref[0] = C / norm


def kernel(anonymized_nodes, walk_masks, kernel):
    k = kernel[:L, :L]
    return pl.pallas_call(
        _cooc_kernel,
        grid=(B,),
        in_specs=[
            pl.BlockSpec((1, W, L), lambda b: (b, 0, 0)),
            pl.BlockSpec((1, W, L), lambda b: (b, 0, 0)),
            pl.BlockSpec((L, L), lambda b: (0, 0)),
        ],
        out_specs=pl.BlockSpec((1, W, W), lambda b: (b, 0, 0)),
        out_shape=jax.ShapeDtypeStruct((B, W, W), jnp.float32),
    )(anonymized_nodes, walk_masks, k)


# speedup vs baseline: 3.5029x; 1.1162x over previous
import jax
import jax.numpy as jnp
from jax import lax
from jax.experimental import pallas as pl
from jax.experimental.pallas import tpu as pltpu

B, W, L = 16, 256, 20
J = L * L


def _cooc_kernel(nodes_ref, masks_ref, k_ref, out_ref,
                 s_ref, kbig_ref, veq_ref, kdrep_ref):
    @pl.when(pl.program_id(0) == 0)
    def _build_constants():
        k = k_ref[...]
        prow = lax.broadcasted_iota(jnp.int32, (L, J), 0)
        jcol = lax.broadcasted_iota(jnp.int32, (L, J), 1)
        S = (prow == jcol // L).astype(jnp.float32)
        s_ref[...] = S.astype(jnp.bfloat16)

        kS = jnp.dot(k, S, preferred_element_type=jnp.float32)
        Krep = lax.dot_general(S, kS, (((0,), (0,)), ((), ())),
                               preferred_element_type=jnp.float32)
        vr = lax.broadcasted_iota(jnp.int32, (J, J), 0) % L
        vc = lax.broadcasted_iota(jnp.int32, (J, J), 1) % L
        veq = vr == vc
        kbig_ref[...] = jnp.where(veq, Krep, 0.0).astype(jnp.bfloat16)
        veq_ref[...] = veq.astype(jnp.float32)

        eyeL = (lax.broadcasted_iota(jnp.int32, (L, L), 0) ==
                lax.broadcasted_iota(jnp.int32, (L, L), 1)).astype(jnp.float32)
        kdiag = jnp.sum(k * eyeL, axis=0, keepdims=True)
        kdrep_ref[...] = jnp.dot(kdiag, S,
                                 preferred_element_type=jnp.float32)

    nodes = nodes_ref[0].astype(jnp.bfloat16)
    masksf = masks_ref[0].astype(jnp.float32)
    S = s_ref[...]

    nrep = jnp.dot(nodes, S, preferred_element_type=jnp.float32)
    mrep = jnp.dot(masksf.astype(jnp.bfloat16), S,
                   preferred_element_type=jnp.float32)
    vidx = (lax.broadcasted_iota(jnp.int32, (W, J), 1) % L).astype(jnp.float32)
    X = jnp.where(nrep == vidx, mrep, 0.0).astype(jnp.bfloat16)

    Y = jnp.dot(X, kbig_ref[...],
                preferred_element_type=jnp.float32)
    C = lax.dot_general(Y.astype(jnp.bfloat16), X,
                        (((1,), (1,)), ((), ())),
                        preferred_element_type=jnp.float32)

    X32 = X.astype(jnp.float32)
    colsum = jnp.sum(X32, axis=0, keepdims=True)
    counts_rep = jnp.dot(colsum, veq_ref[...],
                         preferred_element_type=jnp.float32)
    svec = jnp.where(counts_rep == 1.0, kdrep_ref[...], 0.0)
    contrib = jnp.sum(X32 * svec, axis=1, keepdims=True)

    lens_col = jnp.sum(masksf, axis=1, keepdims=True)
    ones_row = jnp.ones((1, L), dtype=jnp.float32)
    lens_row = lax.dot_general(ones_row, masksf, (((1,), (1,)), ((), ())),
                               preferred_element_type=jnp.float32)
    rl_col = 1.0 / (lens_col + 1e-8)
    rl_row = 1.0 / (lens_row + 1e-8)

    eyeW = (lax.broadcasted_iota(jnp.int32, (W, W), 0) ==
            lax.broadcasted_iota(jnp.int32, (W, W), 1)).astype(jnp.float32)
    out_ref[0] = (C - contrib * eyeW) * (rl_col * rl_row)


def kernel(anonymized_nodes, walk_masks, kernel):
    k = kernel[:L, :L]
    return pl.pallas_call(
        _cooc_kernel,
        grid=(B,),
        in_specs=[
            pl.BlockSpec((1, W, L), lambda b: (b, 0, 0)),
            pl.BlockSpec((1, W, L), lambda b: (b, 0, 0)),
            pl.BlockSpec((L, L), lambda b: (0, 0)),
        ],
        out_specs=pl.BlockSpec((1, W, W), lambda b: (b, 0, 0)),
        out_shape=jax.ShapeDtypeStruct((B, W, W), jnp.float32),
        scratch_shapes=[
            pltpu.VMEM((L, J), jnp.bfloat16),
            pltpu.VMEM((J, J), jnp.bfloat16),
            pltpu.VMEM((J, J), jnp.float32),
            pltpu.VMEM((1, J), jnp.float32),
        ],
    )(anonymized_nodes, walk_masks, k)


# speedup vs baseline: 4.6031x; 1.3141x over previous
import jax
import jax.numpy as jnp
from jax import lax
from jax.experimental import pallas as pl
from jax.experimental.pallas import tpu as pltpu

B, W, L = 16, 256, 20
J = L * L
BB = 4


def _cooc_kernel(nodes_ref, masks_ref, k_ref, out_ref,
                 s_ref, kbig_ref, veq_ref, kdrep_ref):
    @pl.when(pl.program_id(0) == 0)
    def _build_constants():
        k = k_ref[...]
        prow = lax.broadcasted_iota(jnp.int32, (L, J), 0)
        jcol = lax.broadcasted_iota(jnp.int32, (L, J), 1)
        S = (prow == jcol // L).astype(jnp.float32)
        s_ref[...] = S.astype(jnp.bfloat16)

        kS = jnp.dot(k, S, preferred_element_type=jnp.float32)
        Krep = lax.dot_general(S, kS, (((0,), (0,)), ((), ())),
                               preferred_element_type=jnp.float32)
        vr = lax.broadcasted_iota(jnp.int32, (J, J), 0) % L
        vc = lax.broadcasted_iota(jnp.int32, (J, J), 1) % L
        veq = vr == vc
        kbig_ref[...] = jnp.where(veq, Krep, 0.0).astype(jnp.bfloat16)
        veq_ref[...] = veq.astype(jnp.float32)

        eyeL = (lax.broadcasted_iota(jnp.int32, (L, L), 0) ==
                lax.broadcasted_iota(jnp.int32, (L, L), 1)).astype(jnp.float32)
        kdiag = jnp.sum(k * eyeL, axis=0, keepdims=True)
        kdrep_ref[...] = jnp.dot(kdiag, S,
                                 preferred_element_type=jnp.float32)

    S = s_ref[...]
    vidx = (lax.broadcasted_iota(jnp.int32, (W, J), 1) % L).astype(jnp.float32)
    eyeW = (lax.broadcasted_iota(jnp.int32, (W, W), 0) ==
            lax.broadcasted_iota(jnp.int32, (W, W), 1)).astype(jnp.float32)

    for i in range(BB):
        nodes = nodes_ref[i].astype(jnp.bfloat16)
        masksf = masks_ref[i].astype(jnp.float32)

        nrep = jnp.dot(nodes, S, preferred_element_type=jnp.float32)
        mrep = jnp.dot(masksf.astype(jnp.bfloat16), S,
                       preferred_element_type=jnp.float32)
        X = jnp.where(nrep == vidx, mrep, 0.0).astype(jnp.bfloat16)

        lens_col = jnp.sum(masksf, axis=1, keepdims=True)
        rl_col = (1.0 / (lens_col + 1e-8)).astype(jnp.float32)

        Y = jnp.dot(X, kbig_ref[...],
                    preferred_element_type=jnp.float32)
        Yn = (Y * rl_col).astype(jnp.bfloat16)
        Xn = (X.astype(jnp.float32) * rl_col).astype(jnp.bfloat16)
        C = lax.dot_general(Yn, Xn, (((1,), (1,)), ((), ())),
                            preferred_element_type=jnp.float32)

        X32 = X.astype(jnp.float32)
        colsum = jnp.sum(X32, axis=0, keepdims=True)
        counts_rep = jnp.dot(colsum, veq_ref[...],
                             preferred_element_type=jnp.float32)
        svec = jnp.where(counts_rep == 1.0, kdrep_ref[...], 0.0)
        contrib = jnp.sum(X32 * svec, axis=1, keepdims=True)

        out_ref[i] = C - (contrib * rl_col * rl_col) * eyeW


def kernel(anonymized_nodes, walk_masks, kernel):
    k = kernel[:L, :L]
    return pl.pallas_call(
        _cooc_kernel,
        grid=(B // BB,),
        in_specs=[
            pl.BlockSpec((BB, W, L), lambda b: (b, 0, 0)),
            pl.BlockSpec((BB, W, L), lambda b: (b, 0, 0)),
            pl.BlockSpec((L, L), lambda b: (0, 0)),
        ],
        out_specs=pl.BlockSpec((BB, W, W), lambda b: (b, 0, 0)),
        out_shape=jax.ShapeDtypeStruct((B, W, W), jnp.float32),
        scratch_shapes=[
            pltpu.VMEM((L, J), jnp.bfloat16),
            pltpu.VMEM((J, J), jnp.bfloat16),
            pltpu.VMEM((J, J), jnp.float32),
            pltpu.VMEM((1, J), jnp.float32),
        ],
    )(anonymized_nodes, walk_masks, k)
